# baseline (speedup 1.0000x reference)
import functools

import numpy as np
import jax
import jax.numpy as jnp
from jax import lax
from jax.experimental import pallas as pl
from jax.experimental.pallas import tpu as pltpu
from jax.experimental.pallas import tpu_sc as plsc

N = 10000
E = 320000
HID = 128
N_LAYERS = 4
N_CLASSES = 10

NC = 2
NS = 16
NW = NC * NS

CH = 40
NBUF = 4
E_PAD = 327680
E_PER_TILE = E_PAD // NW
CHUNKS = E_PER_TILE // CH
N_PAD = 10112
ROWS_PER_TILE = N_PAD // NS
HIDW = HID // 2

_INV_RHO = np.empty((HID,), np.int32)
for _j in range(4):
    for _k in range(16):
        _INV_RHO[32 * _j + _k] = 32 * _j + 2 * _k
        _INV_RHO[32 * _j + 16 + _k] = 32 * _j + 2 * _k + 1
_MASK_HI = np.int32(-65536)


def _sc_msg_body(hp_hbm, src_hbm, dst_hbm, filt_hbm, out0_hbm, out1_hbm,
                 src_v, dst_v, filt_v, rows_v, sem_in, sem_g, sem_s,
                 agg_shared):
    c = lax.axis_index("c")
    s = lax.axis_index("s")
    wid = c * NS + s
    edge_base = wid * E_PER_TILE

    def issue_loads(g, b):
        base = edge_base + g * CH
        pltpu.async_copy(src_hbm.at[pl.ds(base, CH)], src_v.at[b],
                         sem_in.at[b])
        pltpu.async_copy(dst_hbm.at[wid * CHUNKS + g], dst_v.at[b],
                         sem_in.at[b])
        pltpu.async_copy(filt_hbm.at[pl.ds(base, CH)],
                         filt_v.at[b], sem_in.at[b])

    def wait_loads(g, b):
        base = edge_base + g * CH
        pltpu.make_async_copy(src_hbm.at[pl.ds(base, CH)], src_v.at[b],
                              sem_in.at[b]).wait()
        pltpu.make_async_copy(dst_hbm.at[wid * CHUNKS + g], dst_v.at[b],
                              sem_in.at[b]).wait()
        pltpu.make_async_copy(filt_hbm.at[pl.ds(base, CH)],
                              filt_v.at[b], sem_in.at[b]).wait()

    def issue_gather(b):
        pltpu.async_copy(hp_hbm.at[src_v.at[b]],
                         rows_v.at[pl.ds(b * CH, CH)], sem_g.at[b])

    def wait_gather(b):
        pltpu.make_async_copy(hp_hbm.at[src_v.at[b]],
                              rows_v.at[pl.ds(b * CH, CH)],
                              sem_g.at[b]).wait()

    def issue_scatter(b):
        pltpu.async_copy(rows_v.at[pl.ds(b * CH, CH)],
                         agg_shared.at[dst_v.at[b]], sem_s.at[b], add=True)

    def wait_scatter(b):
        pltpu.make_async_copy(rows_v.at[pl.ds(b * CH, CH)],
                              agg_shared.at[dst_v.at[b]],
                              sem_s.at[b]).wait()

    def mul(b):
        def mul_row(r, _):
            row = b * CH + r
            for j in range(4):
                fv = filt_v[b, r, pl.ds(j * 16, 16)]
                fa = jax.lax.bitcast_convert_type(fv << 16, jnp.float32)
                fb = jax.lax.bitcast_convert_type(fv & _MASK_HI, jnp.float32)
                sa = pl.ds(j * 32, 16)
                sb = pl.ds(j * 32 + 16, 16)
                rows_v[row, sa] = rows_v[row, sa] * fa
                rows_v[row, sb] = rows_v[row, sb] * fb
            return 0

        lax.fori_loop(0, CH, mul_row, 0, unroll=2)

    def emit(g, j, sw=True, l2=True, g1=True):
        b = j % NBUF
        b1 = (j + 1) % NBUF
        b2 = (j + 2) % NBUF
        if sw:
            wait_scatter(b2)
        if l2:
            issue_loads(g + 2, b2)
        if g1:
            wait_loads(g + 1, b1)
            issue_gather(b1)
        wait_gather(b)
        mul(b)
        issue_scatter(b)

    zero16 = jnp.zeros((16,), jnp.float32)

    def zero_row(r, _):
        for j in range(8):
            rows_v[r, pl.ds(j * 16, 16)] = zero16
        return 0

    lax.fori_loop(0, NBUF * CH, zero_row, 0)
    row_base = s * ROWS_PER_TILE
    _full = ROWS_PER_TILE // (NBUF * CH)
    for k in range(_full):
        pltpu.sync_copy(rows_v,
                        agg_shared.at[pl.ds(row_base + k * NBUF * CH,
                                            NBUF * CH)])
    _rem = ROWS_PER_TILE - _full * NBUF * CH
    if _rem:
        pltpu.sync_copy(rows_v.at[pl.ds(0, _rem)],
                        agg_shared.at[pl.ds(row_base + _full * NBUF * CH,
                                            _rem)])
    plsc.subcore_barrier()

    issue_loads(0, 0)
    issue_loads(1, 1)
    wait_loads(0, 0)
    issue_gather(0)
    emit(0, 0, sw=False)
    emit(1, 1, sw=False)
    emit(2, 2)
    emit(3, 3)

    def outer(gg, _):
        g0 = gg * NBUF
        for j in range(NBUF):
            emit(g0 + j, j)
        return 0

    lax.fori_loop(1, CHUNKS // NBUF - 1, outer, 0)

    gl = CHUNKS - NBUF
    emit(gl + 0, 0)
    emit(gl + 1, 1)
    emit(gl + 2, 2, l2=False)
    emit(gl + 3, 3, l2=False, g1=False)
    wait_scatter(2)
    wait_scatter(3)
    plsc.subcore_barrier()

    sl_rows = pl.ds(row_base, ROWS_PER_TILE)

    @pl.when(c == 0)
    def _():
        pltpu.sync_copy(agg_shared.at[sl_rows], out0_hbm.at[sl_rows])

    @pl.when(c == 1)
    def _():
        pltpu.sync_copy(agg_shared.at[sl_rows], out1_hbm.at[sl_rows])


@jax.jit
def _sc_msg(hp, src, dst2d, filt):
    kern = pl.kernel(
        _sc_msg_body,
        out_type=(
            jax.ShapeDtypeStruct((N_PAD, HID), jnp.float32),
            jax.ShapeDtypeStruct((N_PAD, HID), jnp.float32),
        ),
        mesh=plsc.VectorSubcoreMesh(core_axis_name="c", subcore_axis_name="s"),
        scratch_types=[
            pltpu.VMEM((NBUF, CH), jnp.int32),
            pltpu.VMEM((NBUF, CH), jnp.int32),
            pltpu.VMEM((NBUF, CH, HIDW), jnp.int32),
            pltpu.VMEM((NBUF * CH, HID), jnp.float32),
            pltpu.SemaphoreType.DMA((NBUF,)),
            pltpu.SemaphoreType.DMA((NBUF,)),
            pltpu.SemaphoreType.DMA((NBUF,)),
            pltpu.VMEM_SHARED((N_PAD, HID), jnp.float32),
        ],
    )
    return kern(hp, src, dst2d, filt)


BE = 2048


def _filt_body(dist_ref, wf1_ref, bf1_ref, wf2_ref, bf2_ref, out_ref):
    d = dist_ref[0, 0, :]
    centers = lax.broadcasted_iota(jnp.int32, (BE, HID), 1).astype(
        jnp.float32) * (1.0 / (HID - 1))
    dd = d[:, None] - centers
    bf = jnp.exp(dd * dd * (-float(HID)))
    cut = 0.5 * (jnp.cos(jnp.pi * jnp.clip(d, 0.0, 1.0)) + 1.0)
    bf = bf * cut[:, None]
    t = bf @ wf1_ref[...] + bf1_ref[...]
    t = jnp.logaddexp(t, 0.0)
    f = t @ wf2_ref[...] + bf2_ref[...]
    out_ref[...] = f.astype(jnp.bfloat16)


def _filt(dist3d, Wf1, bf1, Wf2, bf2):
    nb = E_PAD // BE
    return pl.pallas_call(
        _filt_body,
        grid=(nb,),
        in_specs=[
            pl.BlockSpec((1, 1, BE), lambda i: (i, 0, 0)),
            pl.BlockSpec((HID, HID), lambda i: (0, 0)),
            pl.BlockSpec((1, HID), lambda i: (0, 0)),
            pl.BlockSpec((HID, HID), lambda i: (0, 0)),
            pl.BlockSpec((1, HID), lambda i: (0, 0)),
        ],
        out_specs=pl.BlockSpec((BE, HID), lambda i: (i, 0)),
        out_shape=jax.ShapeDtypeStruct((E_PAD, HID), jnp.bfloat16),
    )(dist3d, Wf1, bf1, Wf2, bf2)


BN = 1000


def _mm_body(x_ref, w_ref, out_ref):
    out_ref[...] = x_ref[...] @ w_ref[...]


def _mm(x, w):
    return pl.pallas_call(
        _mm_body,
        grid=(N // BN,),
        in_specs=[
            pl.BlockSpec((BN, HID), lambda i: (i, 0)),
            pl.BlockSpec((HID, HID), lambda i: (0, 0)),
        ],
        out_specs=pl.BlockSpec((BN, HID), lambda i: (i, 0)),
        out_shape=jax.ShapeDtypeStruct((N, HID), jnp.float32),
    )(x, w)


def _dense_body(a0_ref, a1_ref, w2_ref, b2_ref, w1n_ref, out_ref):
    h = jnp.maximum((a0_ref[...] + a1_ref[...]) @ w2_ref[...] + b2_ref[...],
                    0.0)
    out_ref[...] = h @ w1n_ref[...]


def _dense_step(a0, a1, W2i, b2i, W1n):
    return pl.pallas_call(
        _dense_body,
        grid=(N // BN,),
        in_specs=[
            pl.BlockSpec((BN, HID), lambda i: (i, 0)),
            pl.BlockSpec((BN, HID), lambda i: (i, 0)),
            pl.BlockSpec((HID, HID), lambda i: (0, 0)),
            pl.BlockSpec((1, HID), lambda i: (0, 0)),
            pl.BlockSpec((HID, HID), lambda i: (0, 0)),
        ],
        out_specs=pl.BlockSpec((BN, HID), lambda i: (i, 0)),
        out_shape=jax.ShapeDtypeStruct((N, HID), jnp.float32),
    )(a0, a1, W2i, b2i, W1n)


def _final_body(a0_ref, a1_ref, w2_ref, b2_ref, wfc_ref, bfc_ref,
                out_ref, acc_ref):
    i = pl.program_id(0)
    h = jnp.maximum((a0_ref[...] + a1_ref[...]) @ w2_ref[...] + b2_ref[...],
                    0.0)
    colsum = jnp.sum(h, axis=0, keepdims=True)

    @pl.when(i == 0)
    def _():
        acc_ref[...] = colsum

    @pl.when(i > 0)
    def _():
        acc_ref[...] = acc_ref[...] + colsum

    @pl.when(i == pl.num_programs(0) - 1)
    def _():
        g = acc_ref[...] * (1.0 / N)
        logits = g @ wfc_ref[...] + bfc_ref[...]
        m = jnp.max(logits, axis=1, keepdims=True)
        z = logits - m
        lse = jnp.log(jnp.sum(jnp.exp(z), axis=1, keepdims=True))
        out_ref[...] = z - lse


def _final(a0, a1, W2i, b2i, Wfc, bfc):
    return pl.pallas_call(
        _final_body,
        grid=(N // BN,),
        in_specs=[
            pl.BlockSpec((BN, HID), lambda i: (i, 0)),
            pl.BlockSpec((BN, HID), lambda i: (i, 0)),
            pl.BlockSpec((HID, HID), lambda i: (0, 0)),
            pl.BlockSpec((1, HID), lambda i: (0, 0)),
            pl.BlockSpec((HID, N_CLASSES), lambda i: (0, 0)),
            pl.BlockSpec((1, N_CLASSES), lambda i: (0, 0)),
        ],
        out_specs=pl.BlockSpec((1, N_CLASSES), lambda i: (0, 0)),
        out_shape=jax.ShapeDtypeStruct((1, N_CLASSES), jnp.float32),
        scratch_shapes=[pltpu.VMEM((1, HID), jnp.float32)],
    )(a0, a1, W2i, b2i, Wfc, bfc)


def kernel(x, edge_index, edge_dist, W1, W2, b2, Wf1, bf1, Wf2, bf2, Wfc, bfc):
    src = edge_index[0]
    dst = edge_index[1]
    pad = E_PAD - E
    src_p = jnp.concatenate([src, jnp.zeros((pad,), jnp.int32)])
    dst_p = jnp.concatenate([dst, jnp.full((pad,), N, jnp.int32)])
    dst2d = dst_p.reshape(E_PAD // CH, CH)
    dist_p = jnp.concatenate([edge_dist, jnp.zeros((pad,), jnp.float32)])
    dist3d = dist_p.reshape(E_PAD // BE, 1, BE)

    filt = jax.lax.bitcast_convert_type(
        _filt(dist3d, Wf1, bf1.reshape(1, HID), Wf2,
              bf2.reshape(1, HID)).reshape(E_PAD, HIDW, 2), jnp.int32)

    W2p = W2[:, _INV_RHO, :]

    W1p = W1[:, :, _INV_RHO]

    hp = _mm(x, W1p[0])
    for i in range(N_LAYERS):
        a0, a1 = _sc_msg(hp, src_p, dst2d, filt)
        a0 = a0[:N]
        a1 = a1[:N]
        if i < N_LAYERS - 1:
            hp = _dense_step(a0, a1, W2p[i], b2[i].reshape(1, HID),
                             W1p[i + 1])
        else:
            out = _final(a0, a1, W2p[i], b2[i].reshape(1, HID),
                         Wfc, bfc.reshape(1, N_CLASSES))
    return out


# speedup vs baseline: 1.0079x; 1.0079x over previous
import numpy as np
import jax
import jax.numpy as jnp
from jax import lax
from jax.experimental import pallas as pl
from jax.experimental.pallas import tpu as pltpu
from jax.experimental.pallas import tpu_sc as plsc

N = 10000
E = 320000
HID = 128
N_LAYERS = 4
N_CLASSES = 10

NC = 2
NS = 16
NW = NC * NS
CPT = HID // NW

CHE = 2048
E_PAD = 327680
CHUNKS = E_PAD // CHE
N_TC = 10240
BNT = 1280

_MASK_HI = np.int32(-65536)
_MASK_LO = np.int32(0xFFFF)


def _sc_msg_body(hpT_hbm, pk_hbm, filtT_hbm, outT_hbm,
                 idx_v, filt_v, hp_v, agg_v, sem_i, sem_f):
    c = lax.axis_index("c")
    s = lax.axis_index("s")
    wid = c * NS + s

    zero16 = jnp.zeros((16,), jnp.float32)

    pltpu.sync_copy(hpT_hbm.at[wid], hp_v)

    def zcol(w, _):
        agg_v[pl.ds(w * 16, 16)] = zero16
        return 0
    lax.fori_loop(0, CPT * N_TC // 16, zcol, 0, unroll=4)

    def issue_loads(g, b):
        pltpu.async_copy(pk_hbm.at[pl.ds(g * CHE, CHE)], idx_v.at[b],
                         sem_i.at[b])
        pltpu.async_copy(filtT_hbm.at[wid, :, pl.ds(g * CHE, CHE)],
                         filt_v.at[b], sem_f.at[b])

    def wait_loads(g, b):
        pltpu.make_async_copy(pk_hbm.at[pl.ds(g * CHE, CHE)], idx_v.at[b],
                              sem_i.at[b]).wait()
        pltpu.make_async_copy(filtT_hbm.at[wid, :, pl.ds(g * CHE, CHE)],
                              filt_v.at[b], sem_f.at[b]).wait()

    def compute(b):
        def step(i, _):
            pk = idx_v[b, pl.ds(i * 16, 16)]
            sidx = pk & _MASK_LO
            didx = lax.shift_right_logical(pk, 16)
            for c4 in range(CPT):
                off = np.int32(c4 * N_TC)
                fw = filt_v[b, c4, pl.ds(i * 16, 16)]
                g16 = plsc.load_gather(hp_v, [sidx + off])
                plsc.addupdate_scatter(agg_v, [didx + off], g16 * fw)
            return 0

        lax.fori_loop(0, CHE // 16, step, 0)

    issue_loads(0, 0)
    issue_loads(1, 1)

    def outer(gg, _):
        for b in range(2):
            g = gg * 2 + b
            wait_loads(g, b)
            compute(b)

            @pl.when(g + 2 < CHUNKS)
            def _():
                issue_loads(g + 2, b)
        return 0

    lax.fori_loop(0, CHUNKS // 2, outer, 0)

    pltpu.sync_copy(agg_v, outT_hbm.at[wid])


@jax.jit
def _sc_msg(hpT3, pk, filtT3):
    kern = pl.kernel(
        _sc_msg_body,
        out_type=jax.ShapeDtypeStruct((NW, CPT * N_TC), jnp.float32),
        mesh=plsc.VectorSubcoreMesh(core_axis_name="c", subcore_axis_name="s"),
        compiler_params=pltpu.CompilerParams(needs_layout_passes=False),
        scratch_types=[
            pltpu.VMEM((2, CHE), jnp.int32),
            pltpu.VMEM((2, CPT, CHE), jnp.float32),
            pltpu.VMEM((CPT * N_TC,), jnp.float32),
            pltpu.VMEM((CPT * N_TC,), jnp.float32),
            pltpu.SemaphoreType.DMA((2,)),
            pltpu.SemaphoreType.DMA((2,)),
        ],
    )
    return kern(hpT3, pk, filtT3)


BE = 2048


def _filt_body(dist_ref, wf1_ref, bf1_ref, wf2_ref, bf2_ref, out_ref):
    d = dist_ref[0, 0, :]
    centers = lax.broadcasted_iota(jnp.int32, (BE, HID), 1).astype(
        jnp.float32) * (1.0 / (HID - 1))
    dd = d[:, None] - centers
    bf = jnp.exp(dd * dd * (-float(HID)))
    cut = 0.5 * (jnp.cos(jnp.pi * jnp.clip(d, 0.0, 1.0)) + 1.0)
    bf = bf * cut[:, None]
    t = bf @ wf1_ref[...] + bf1_ref[...]
    t = jnp.logaddexp(t, 0.0)
    f = t @ wf2_ref[...] + bf2_ref[...]
    out_ref[...] = f.T


def _filt(dist3d, Wf1, bf1, Wf2, bf2):
    nb = E_PAD // BE
    return pl.pallas_call(
        _filt_body,
        grid=(nb,),
        in_specs=[
            pl.BlockSpec((1, 1, BE), lambda i: (i, 0, 0)),
            pl.BlockSpec((HID, HID), lambda i: (0, 0)),
            pl.BlockSpec((1, HID), lambda i: (0, 0)),
            pl.BlockSpec((HID, HID), lambda i: (0, 0)),
            pl.BlockSpec((1, HID), lambda i: (0, 0)),
        ],
        out_specs=pl.BlockSpec((HID, BE), lambda i: (0, i)),
        out_shape=jax.ShapeDtypeStruct((HID, E_PAD), jnp.float32),
    )(dist3d, Wf1, bf1, Wf2, bf2)


def _mmT_body(w_ref, xT_ref, out_ref):
    out_ref[...] = jnp.dot(w_ref[...].T, xT_ref[...])


def _mmT(w, xT):
    return pl.pallas_call(
        _mmT_body,
        grid=(N_TC // BNT,),
        in_specs=[
            pl.BlockSpec((HID, HID), lambda i: (0, 0)),
            pl.BlockSpec((HID, BNT), lambda i: (0, i)),
        ],
        out_specs=pl.BlockSpec((HID, BNT), lambda i: (0, i)),
        out_shape=jax.ShapeDtypeStruct((HID, N_TC), jnp.float32),
    )(w, xT)


def _denseT_body(a_ref, w2_ref, b2_ref, w1n_ref, out_ref):
    h = jnp.maximum(jnp.dot(w2_ref[...].T, a_ref[...]) + b2_ref[...],
                    0.0)
    out_ref[...] = jnp.dot(w1n_ref[...].T, h)


def _denseT(aT, W2i, b2i, W1n):
    return pl.pallas_call(
        _denseT_body,
        grid=(N_TC // BNT,),
        in_specs=[
            pl.BlockSpec((HID, BNT), lambda i: (0, i)),
            pl.BlockSpec((HID, HID), lambda i: (0, 0)),
            pl.BlockSpec((HID, 1), lambda i: (0, 0)),
            pl.BlockSpec((HID, HID), lambda i: (0, 0)),
        ],
        out_specs=pl.BlockSpec((HID, BNT), lambda i: (0, i)),
        out_shape=jax.ShapeDtypeStruct((HID, N_TC), jnp.float32),
    )(aT, W2i, b2i, W1n)


def _finalT_body(a_ref, w2_ref, b2_ref, wfc_ref, bfc_ref, out_ref, acc_ref):
    i = pl.program_id(0)
    h = jnp.maximum(jnp.dot(w2_ref[...].T, a_ref[...]) + b2_ref[...],
                    0.0)
    col = lax.broadcasted_iota(jnp.int32, (HID, BNT), 1) + i * BNT
    h = jnp.where(col < N, h, 0.0)
    rowsum = jnp.sum(h, axis=1, keepdims=True)

    @pl.when(i == 0)
    def _():
        acc_ref[...] = rowsum

    @pl.when(i > 0)
    def _():
        acc_ref[...] = acc_ref[...] + rowsum

    @pl.when(i == pl.num_programs(0) - 1)
    def _():
        g = acc_ref[...].T * (1.0 / N)
        logits = g @ wfc_ref[...] + bfc_ref[...]
        m = jnp.max(logits, axis=1, keepdims=True)
        z = logits - m
        lse = jnp.log(jnp.sum(jnp.exp(z), axis=1, keepdims=True))
        out_ref[...] = z - lse


def _finalT(aT, W2i, b2i, Wfc, bfc):
    return pl.pallas_call(
        _finalT_body,
        grid=(N_TC // BNT,),
        in_specs=[
            pl.BlockSpec((HID, BNT), lambda i: (0, i)),
            pl.BlockSpec((HID, HID), lambda i: (0, 0)),
            pl.BlockSpec((HID, 1), lambda i: (0, 0)),
            pl.BlockSpec((HID, N_CLASSES), lambda i: (0, 0)),
            pl.BlockSpec((1, N_CLASSES), lambda i: (0, 0)),
        ],
        out_specs=pl.BlockSpec((1, N_CLASSES), lambda i: (0, 0)),
        out_shape=jax.ShapeDtypeStruct((1, N_CLASSES), jnp.float32),
        scratch_shapes=[pltpu.VMEM((HID, 1), jnp.float32)],
    )(aT, W2i, b2i, Wfc, bfc)


def kernel(x, edge_index, edge_dist, W1, W2, b2, Wf1, bf1, Wf2, bf2, Wfc, bfc):
    src = edge_index[0]
    dst = edge_index[1]
    pad = E_PAD - E
    src_p = jnp.concatenate([src, jnp.zeros((pad,), jnp.int32)])
    dst_p = jnp.concatenate([dst, jnp.full((pad,), N, jnp.int32)])
    pk = src_p + dst_p * 65536
    dist_p = jnp.concatenate([edge_dist, jnp.zeros((pad,), jnp.float32)])
    dist3d = dist_p.reshape(E_PAD // BE, 1, BE)

    filtT3 = _filt(dist3d, Wf1, bf1.reshape(1, HID), Wf2,
                   bf2.reshape(1, HID)).reshape(NW, CPT, E_PAD)

    xT = jnp.pad(x.T, ((0, 0), (0, N_TC - N)))
    hpT = _mmT(W1[0], xT)
    b2c = b2.reshape(N_LAYERS, HID, 1)
    for i in range(N_LAYERS):
        aggT3 = _sc_msg(hpT.reshape(NW, CPT * N_TC), pk, filtT3)
        aT = aggT3.reshape(HID, N_TC)
        if i < N_LAYERS - 1:
            hpT = _denseT(aT, W2[i], b2c[i], W1[i + 1])
        else:
            out = _finalT(aT, W2[i], b2c[i], Wfc, bfc.reshape(1, N_CLASSES))
    return out


# speedup vs baseline: 2.0808x; 2.0646x over previous
import numpy as np
import jax
import jax.numpy as jnp
from jax import lax
from jax.experimental import pallas as pl
from jax.experimental.pallas import tpu as pltpu
from jax.experimental.pallas import tpu_sc as plsc

N = 10000
E = 320000
HID = 128
N_LAYERS = 4
N_CLASSES = 10

NC = 2
NS = 16
NW = NC * NS
CPT = HID // NW

CHE = 2048
E_PAD = 327680
CHUNKS = E_PAD // CHE
N_TC = 10240
BNT = 1280

_MASK_HI = np.int32(-65536)
_MASK_LO = np.int32(0xFFFF)


def _sc_msg_body(hpT_hbm, pk_hbm, filtT_hbm, outT_hbm,
                 idx_v, filt_v, hp_v, agg_v, sem_i, sem_f):
    c = lax.axis_index("c")
    s = lax.axis_index("s")
    wid = c * NS + s

    zero16 = jnp.zeros((16,), jnp.float32)

    pltpu.sync_copy(hpT_hbm.at[wid], hp_v)

    def zcol(w, _):
        agg_v[pl.ds(w * 16, 16)] = zero16
        return 0
    lax.fori_loop(0, CPT * N_TC // 16, zcol, 0, unroll=4)

    def issue_loads(g, b):
        pltpu.async_copy(pk_hbm.at[pl.ds(g * CHE, CHE)], idx_v.at[b],
                         sem_i.at[b])
        pltpu.async_copy(filtT_hbm.at[wid, :, pl.ds(g * CHE, CHE)],
                         filt_v.at[b], sem_f.at[b])

    def wait_loads(g, b):
        pltpu.make_async_copy(pk_hbm.at[pl.ds(g * CHE, CHE)], idx_v.at[b],
                              sem_i.at[b]).wait()
        pltpu.make_async_copy(filtT_hbm.at[wid, :, pl.ds(g * CHE, CHE)],
                              filt_v.at[b], sem_f.at[b]).wait()

    def compute(b):
        @plsc.parallel_loop(0, CHE // 16, unroll=8)
        def _edge_step(i):
            pk = idx_v[b, pl.ds(i * 16, 16)]
            sidx = pk & _MASK_LO
            didx = lax.shift_right_logical(pk, 16)
            for c4 in range(CPT):
                off = np.int32(c4 * N_TC)
                fw = filt_v[b, c4, pl.ds(i * 16, 16)]
                g16 = plsc.load_gather(hp_v, [sidx + off])
                plsc.addupdate_scatter(agg_v, [didx + off], g16 * fw)

    issue_loads(0, 0)
    issue_loads(1, 1)

    def outer(gg, _):
        for b in range(2):
            g = gg * 2 + b
            wait_loads(g, b)
            compute(b)

            @pl.when(g + 2 < CHUNKS)
            def _():
                issue_loads(g + 2, b)
        return 0

    lax.fori_loop(0, CHUNKS // 2, outer, 0)

    pltpu.sync_copy(agg_v, outT_hbm.at[wid])


@jax.jit
def _sc_msg(hpT3, pk, filtT3):
    kern = pl.kernel(
        _sc_msg_body,
        out_type=jax.ShapeDtypeStruct((NW, CPT * N_TC), jnp.float32),
        mesh=plsc.VectorSubcoreMesh(core_axis_name="c", subcore_axis_name="s"),
        compiler_params=pltpu.CompilerParams(needs_layout_passes=False),
        scratch_types=[
            pltpu.VMEM((2, CHE), jnp.int32),
            pltpu.VMEM((2, CPT, CHE), jnp.float32),
            pltpu.VMEM((CPT * N_TC,), jnp.float32),
            pltpu.VMEM((CPT * N_TC,), jnp.float32),
            pltpu.SemaphoreType.DMA((2,)),
            pltpu.SemaphoreType.DMA((2,)),
        ],
    )
    return kern(hpT3, pk, filtT3)


BE = 2048


def _filt_body(dist_ref, wf1_ref, bf1_ref, wf2_ref, bf2_ref, out_ref):
    d = dist_ref[0, 0, :]
    centers = lax.broadcasted_iota(jnp.int32, (BE, HID), 1).astype(
        jnp.float32) * (1.0 / (HID - 1))
    dd = d[:, None] - centers
    bf = jnp.exp(dd * dd * (-float(HID)))
    cut = 0.5 * (jnp.cos(jnp.pi * jnp.clip(d, 0.0, 1.0)) + 1.0)
    bf = bf * cut[:, None]
    t = bf @ wf1_ref[...] + bf1_ref[...]
    t = jnp.logaddexp(t, 0.0)
    f = t @ wf2_ref[...] + bf2_ref[...]
    out_ref[...] = f.T


def _filt(dist3d, Wf1, bf1, Wf2, bf2):
    nb = E_PAD // BE
    return pl.pallas_call(
        _filt_body,
        grid=(nb,),
        in_specs=[
            pl.BlockSpec((1, 1, BE), lambda i: (i, 0, 0)),
            pl.BlockSpec((HID, HID), lambda i: (0, 0)),
            pl.BlockSpec((1, HID), lambda i: (0, 0)),
            pl.BlockSpec((HID, HID), lambda i: (0, 0)),
            pl.BlockSpec((1, HID), lambda i: (0, 0)),
        ],
        out_specs=pl.BlockSpec((HID, BE), lambda i: (0, i)),
        out_shape=jax.ShapeDtypeStruct((HID, E_PAD), jnp.float32),
    )(dist3d, Wf1, bf1, Wf2, bf2)


def _mmT_body(w_ref, xT_ref, out_ref):
    out_ref[...] = jnp.dot(w_ref[...].T, xT_ref[...])


def _mmT(w, xT):
    return pl.pallas_call(
        _mmT_body,
        grid=(N_TC // BNT,),
        in_specs=[
            pl.BlockSpec((HID, HID), lambda i: (0, 0)),
            pl.BlockSpec((HID, BNT), lambda i: (0, i)),
        ],
        out_specs=pl.BlockSpec((HID, BNT), lambda i: (0, i)),
        out_shape=jax.ShapeDtypeStruct((HID, N_TC), jnp.float32),
    )(w, xT)


def _denseT_body(a_ref, w2_ref, b2_ref, w1n_ref, out_ref):
    h = jnp.maximum(jnp.dot(w2_ref[...].T, a_ref[...]) + b2_ref[...],
                    0.0)
    out_ref[...] = jnp.dot(w1n_ref[...].T, h)


def _denseT(aT, W2i, b2i, W1n):
    return pl.pallas_call(
        _denseT_body,
        grid=(N_TC // BNT,),
        in_specs=[
            pl.BlockSpec((HID, BNT), lambda i: (0, i)),
            pl.BlockSpec((HID, HID), lambda i: (0, 0)),
            pl.BlockSpec((HID, 1), lambda i: (0, 0)),
            pl.BlockSpec((HID, HID), lambda i: (0, 0)),
        ],
        out_specs=pl.BlockSpec((HID, BNT), lambda i: (0, i)),
        out_shape=jax.ShapeDtypeStruct((HID, N_TC), jnp.float32),
    )(aT, W2i, b2i, W1n)


def _finalT_body(a_ref, w2_ref, b2_ref, wfc_ref, bfc_ref, out_ref, acc_ref):
    i = pl.program_id(0)
    h = jnp.maximum(jnp.dot(w2_ref[...].T, a_ref[...]) + b2_ref[...],
                    0.0)
    col = lax.broadcasted_iota(jnp.int32, (HID, BNT), 1) + i * BNT
    h = jnp.where(col < N, h, 0.0)
    rowsum = jnp.sum(h, axis=1, keepdims=True)

    @pl.when(i == 0)
    def _():
        acc_ref[...] = rowsum

    @pl.when(i > 0)
    def _():
        acc_ref[...] = acc_ref[...] + rowsum

    @pl.when(i == pl.num_programs(0) - 1)
    def _():
        g = acc_ref[...].T * (1.0 / N)
        logits = g @ wfc_ref[...] + bfc_ref[...]
        m = jnp.max(logits, axis=1, keepdims=True)
        z = logits - m
        lse = jnp.log(jnp.sum(jnp.exp(z), axis=1, keepdims=True))
        out_ref[...] = z - lse


def _finalT(aT, W2i, b2i, Wfc, bfc):
    return pl.pallas_call(
        _finalT_body,
        grid=(N_TC // BNT,),
        in_specs=[
            pl.BlockSpec((HID, BNT), lambda i: (0, i)),
            pl.BlockSpec((HID, HID), lambda i: (0, 0)),
            pl.BlockSpec((HID, 1), lambda i: (0, 0)),
            pl.BlockSpec((HID, N_CLASSES), lambda i: (0, 0)),
            pl.BlockSpec((1, N_CLASSES), lambda i: (0, 0)),
        ],
        out_specs=pl.BlockSpec((1, N_CLASSES), lambda i: (0, 0)),
        out_shape=jax.ShapeDtypeStruct((1, N_CLASSES), jnp.float32),
        scratch_shapes=[pltpu.VMEM((HID, 1), jnp.float32)],
    )(aT, W2i, b2i, Wfc, bfc)


def kernel(x, edge_index, edge_dist, W1, W2, b2, Wf1, bf1, Wf2, bf2, Wfc, bfc):
    src = edge_index[0]
    dst = edge_index[1]
    pad = E_PAD - E
    src_p = jnp.concatenate([src, jnp.zeros((pad,), jnp.int32)])
    dst_p = jnp.concatenate([dst, jnp.full((pad,), N, jnp.int32)])
    pk = src_p + dst_p * 65536
    dist_p = jnp.concatenate([edge_dist, jnp.zeros((pad,), jnp.float32)])
    dist3d = dist_p.reshape(E_PAD // BE, 1, BE)

    filtT3 = _filt(dist3d, Wf1, bf1.reshape(1, HID), Wf2,
                   bf2.reshape(1, HID)).reshape(NW, CPT, E_PAD)

    xT = jnp.pad(x.T, ((0, 0), (0, N_TC - N)))
    hpT = _mmT(W1[0], xT)
    b2c = b2.reshape(N_LAYERS, HID, 1)
    for i in range(N_LAYERS):
        aggT3 = _sc_msg(hpT.reshape(NW, CPT * N_TC), pk, filtT3)
        aT = aggT3.reshape(HID, N_TC)
        if i < N_LAYERS - 1:
            hpT = _denseT(aT, W2[i], b2c[i], W1[i + 1])
        else:
            out = _finalT(aT, W2[i], b2c[i], Wfc, bfc.reshape(1, N_CLASSES))
    return out
